# baseline (speedup 1.0000x reference)
import jax
import jax.numpy as jnp
from jax import lax
from jax.experimental import pallas as pl
from jax.experimental.pallas import tpu as pltpu
from jax.experimental.pallas import tpu_sc as plsc

B = 4096
SEQ = 200
D = 32
OUT_DIM = 128

NC = 2
NS = 16
NW = NC * NS
RPW = B // NW
IPW = RPW * SEQ
UNIT = 128
UPC = 10
CHUNK_ROWS = UPC * UNIT
CHUNKS = IPW // CHUNK_ROWS


def _sc_body(x_hbm, table_hbm, out_hbm,
             idx_a, idx_b, dest_a, dest_b, rows_a, rows_b, pooled_v,
             accum_sh, gs_a, gs_b, ss_a, ss_b):
    c = lax.axis_index("c")
    s = lax.axis_index("s")
    wid = s * NC + c

    z = jnp.zeros((16,), jnp.float32)
    for r in range(RPW):
        rows_a[r, 0:16] = z
        rows_a[r, 16:32] = z
    pltpu.sync_copy(rows_a.at[pl.ds(0, RPW)],
                    accum_sh.at[0].at[pl.ds(s * RPW, RPW)])
    pltpu.sync_copy(rows_a.at[pl.ds(0, RPW)],
                    accum_sh.at[1].at[pl.ds(s * RPW, RPW)])

    base0 = wid * IPW
    lane = lax.iota(jnp.int32, 16)
    srow = s * RPW

    bufs = ((idx_a, dest_a, rows_a, gs_a, ss_a),
            (idx_b, dest_b, rows_b, gs_b, ss_b))

    def stage(i, bf):
        idx_v, dest_v = bf[0], bf[1]
        pltpu.sync_copy(x_hbm.at[pl.ds(base0 + i * CHUNK_ROWS, CHUNK_ROWS)],
                        idx_v)
        for u in range(UPC):
            for k in range(UNIT // 16):
                f = i * CHUNK_ROWS + u * UNIT + k * 16
                dest_v[u, k * 16:(k + 1) * 16] = srow + lax.div(f + lane, SEQ)

    def fire_g(bf):
        idx_v, rows_v, gsem = bf[0], bf[2], bf[3]
        for u in range(UPC):
            pltpu.async_copy(table_hbm.at[idx_v.at[pl.ds(u * UNIT, UNIT)]],
                             rows_v.at[pl.ds(u * UNIT, UNIT)], gsem)

    def wait_g(bf):
        idx_v, rows_v, gsem = bf[0], bf[2], bf[3]
        for u in range(UPC):
            pltpu.make_async_copy(
                table_hbm.at[idx_v.at[pl.ds(u * UNIT, UNIT)]],
                rows_v.at[pl.ds(u * UNIT, UNIT)], gsem).wait()

    def fire_s(bf):
        dest_v, rows_v, ssem = bf[1], bf[2], bf[4]
        for u in range(UPC):
            pltpu.async_copy(rows_v.at[pl.ds(u * UNIT, UNIT)],
                             accum_sh.at[u % 2].at[dest_v.at[u]], ssem,
                             add=True)

    def wait_s(bf):
        dest_v, rows_v, ssem = bf[1], bf[2], bf[4]
        for u in range(UPC):
            pltpu.make_async_copy(rows_v.at[pl.ds(u * UNIT, UNIT)],
                                  accum_sh.at[u % 2].at[dest_v.at[u]],
                                  ssem).wait()

    stage(0, bufs[0])
    fire_g(bufs[0])
    stage(1, bufs[1])
    fire_g(bufs[1])
    for i in range(CHUNKS):
        bf = bufs[i % 2]
        wait_g(bf)
        fire_s(bf)
        wait_s(bf)
        if i + 2 < CHUNKS:
            stage(i + 2, bf)
            fire_g(bf)

    pltpu.sync_copy(accum_sh.at[0].at[pl.ds(s * RPW, RPW)], pooled_v)
    pltpu.sync_copy(accum_sh.at[1].at[pl.ds(s * RPW, RPW)],
                    rows_a.at[pl.ds(0, RPW)])
    for r in range(RPW):
        pooled_v[r, 0:16] = pooled_v[r, 0:16] + rows_a[r, 0:16]
        pooled_v[r, 16:32] = pooled_v[r, 16:32] + rows_a[r, 16:32]
    pltpu.sync_copy(pooled_v, out_hbm.at[pl.ds(wid * RPW, RPW)])


def _sc_pooled_sums(x1, table):
    mesh = plsc.VectorSubcoreMesh(core_axis_name="c", subcore_axis_name="s",
                                  num_cores=NC, num_subcores=NS)
    return pl.kernel(
        _sc_body,
        out_type=jax.ShapeDtypeStruct((B, D), jnp.float32),
        mesh=mesh,
        scratch_types=[
            pltpu.VMEM((CHUNK_ROWS,), jnp.int32),
            pltpu.VMEM((CHUNK_ROWS,), jnp.int32),
            pltpu.VMEM((UPC, UNIT), jnp.int32),
            pltpu.VMEM((UPC, UNIT), jnp.int32),
            pltpu.VMEM((CHUNK_ROWS, D), jnp.float32),
            pltpu.VMEM((CHUNK_ROWS, D), jnp.float32),
            pltpu.VMEM((RPW, D), jnp.float32),
            pltpu.VMEM_SHARED((2, NS * RPW, D), jnp.float32),
            pltpu.SemaphoreType.DMA,
            pltpu.SemaphoreType.DMA,
            pltpu.SemaphoreType.DMA,
            pltpu.SemaphoreType.DMA,
        ],
        compiler_params=pltpu.CompilerParams(use_tc_tiling_on_sc=False),
    )(x1, table)


def _mm_body(p_ref, w_ref, b_ref, o_ref):
    p = p_ref[...] * (1.0 / SEQ)
    o_ref[...] = lax.dot_general(
        p, w_ref[...], (((1,), (1,)), ((), ())),
        preferred_element_type=jnp.float32) + b_ref[...]


def _classifier(pooled_sums, W, b):
    return pl.pallas_call(
        _mm_body,
        out_shape=jax.ShapeDtypeStruct((B, OUT_DIM), jnp.float32),
    )(pooled_sums, W, b.reshape(1, OUT_DIM))


def kernel(x, emb_table, W, b):
    x1 = x.astype(jnp.int32).reshape(B * SEQ)
    pooled_sums = _sc_pooled_sums(x1, emb_table)
    return _classifier(pooled_sums, W, b)
